# Initial kernel scaffold; baseline (speedup 1.0000x reference)
import jax
import jax.numpy as jnp
from jax.experimental import pallas as pl


def kernel(x, edge_index, batch, s, W_pre, b_pre, W_se, b_se, W1, W2, gamma, beta, Wg, bg, Whp, b_hp, Wf1, Wf2, Wf3, Wsc):
    raise NotImplementedError("write your pallas kernel here")


# speedup vs baseline: 2.9760x; 2.9760x over previous
import functools

import jax
import jax.numpy as jnp
from jax import lax
from jax.experimental import pallas as pl
from jax.experimental.pallas import tpu as pltpu
from jax.experimental.pallas import tpu_sc as plsc

N = 10000
E = 320000
D = 128
G = 128
LAYERS = 3

NC = 2
NS = 16
NW = NC * NS
CHW = 128
EW = E // NW
CH = (EW + CHW - 1) // CHW
CH = CH + (CH % 2)
EWP = CH * CHW
NP = 10112
RPT = NP // NS


@functools.lru_cache(maxsize=None)
def _get_mesh():
    return plsc.VectorSubcoreMesh(
        core_axis_name="c", subcore_axis_name="s", num_cores=NC, num_subcores=NS
    )

def _sc_scatter_body(tab_hbm, srcw_hbm, dstw_hbm, zeros_hbm, out_hbm,
                     src_v, dst_v, rows_v, acc_sh, sem):
    c = lax.axis_index("c")
    t = lax.axis_index("s")
    wid = c * NS + t
    pltpu.sync_copy(srcw_hbm.at[wid], src_v)
    pltpu.sync_copy(dstw_hbm.at[wid], dst_v)
    r0 = t * RPT
    pltpu.sync_copy(zeros_hbm.at[pl.ds(r0, RPT)], acc_sh.at[pl.ds(r0, RPT)])
    plsc.subcore_barrier()

    def body(j, carry):
        pltpu.async_copy(tab_hbm.at[src_v.at[j]], rows_v, sem).wait()
        pltpu.sync_copy(rows_v, acc_sh.at[dst_v.at[j]], add=True)
        return carry

    lax.fori_loop(0, CH, body, 0)
    plsc.subcore_barrier()
    pltpu.sync_copy(acc_sh.at[pl.ds(r0, RPT)], out_hbm.at[c].at[pl.ds(r0, RPT)])


@functools.lru_cache(maxsize=None)
def _get_sc_scatter():
    return pl.kernel(
        _sc_scatter_body,
        out_type=jax.ShapeDtypeStruct((NC, NP, D), jnp.float32),
        mesh=_get_mesh(),
        scratch_types=[
            pltpu.VMEM((CH, CHW), jnp.int32),
            pltpu.VMEM((CH, CHW), jnp.int32),
            pltpu.VMEM((CHW, D), jnp.float32),
            pltpu.VMEM_SHARED((NP, D), jnp.float32),
            pltpu.SemaphoreType.DMA,
        ],
    )


def _leaky(h):
    return jnp.where(h > 0, h, h * 0.01)


def _bdot(a, b):
    return jnp.dot(a.astype(jnp.bfloat16), b.astype(jnp.bfloat16),
                   preferred_element_type=jnp.float32)


RBLK = NP // 8


def _tc_pre_body(x_ref, wpre_ref, bpre_ref, s_ref, wse_ref, bse_ref,
                 wg0_ref, degp_ref, x1_ref, s1_ref, t1_ref, dinv_ref):
    i = pl.program_id(0)
    rows = i * RBLK + lax.broadcasted_iota(jnp.int32, (RBLK, 1), 0)
    zmask = (rows < N).astype(jnp.float32)
    x1 = _bdot(x_ref[...], wpre_ref[...]) + bpre_ref[...]
    s1 = _bdot(s_ref[...], wse_ref[...]) + bse_ref[...]
    deg = degp_ref[0, :, 0:1] + degp_ref[1, :, 0:1] + 1.0
    dinv = lax.rsqrt(deg)
    x1_ref[...] = x1 * zmask
    s1 = s1 * zmask
    s1_ref[...] = s1
    t1_ref[...] = _bdot(s1, wg0_ref[...]) * dinv * zmask
    dinv_ref[...] = dinv


def _tc_pre(xp, wpre, bpre, sp, wse, bse, wg0, degp):
    row_spec = pl.BlockSpec((RBLK, D), lambda i: (i, 0))
    return pl.pallas_call(
        _tc_pre_body,
        grid=(NP // RBLK,),
        in_specs=[
            row_spec,
            pl.BlockSpec((D, D), lambda i: (0, 0)),
            pl.BlockSpec((1, D), lambda i: (0, 0)),
            pl.BlockSpec((RBLK, 16), lambda i: (i, 0)),
            pl.BlockSpec((16, D), lambda i: (0, 0)),
            pl.BlockSpec((1, D), lambda i: (0, 0)),
            pl.BlockSpec((D, D), lambda i: (0, 0)),
            pl.BlockSpec((NC, RBLK, D), lambda i: (0, i, 0)),
        ],
        out_specs=[row_spec, row_spec, row_spec,
                   pl.BlockSpec((RBLK, 1), lambda i: (i, 0))],
        out_shape=[
            jax.ShapeDtypeStruct((NP, D), jnp.float32),
            jax.ShapeDtypeStruct((NP, D), jnp.float32),
            jax.ShapeDtypeStruct((NP, D), jnp.float32),
            jax.ShapeDtypeStruct((NP, 1), jnp.float32),
        ],
    )(xp, wpre, bpre, sp, wse, bse, wg0, degp)


def _tc_layer_body(x_ref, s_ref, t_ref, dinv_ref, px_ref, ps_ref, pt_ref,
                   w1_ref, w2_ref, wgn_ref, bg_ref,
                   xn_ref, sn_ref, tn_ref):
    i = pl.program_id(0)
    x = x_ref[...]
    s = s_ref[...]
    t = t_ref[...]
    dinv = dinv_ref[...]
    hx = x + px_ref[0] + px_ref[1]
    hs = s + ps_ref[0] + ps_ref[1]
    h = _bdot(hx, w1_ref[:D, :]) + _bdot(hs, w1_ref[D:, :])
    h = _leaky(h)
    xn = _bdot(h, w2_ref[...])
    aggt = pt_ref[0] + pt_ref[1]
    sn = jnp.tanh((aggt + t) * dinv + bg_ref[...])
    rows = i * RBLK + lax.broadcasted_iota(jnp.int32, (RBLK, 1), 0)
    zmask = (rows < N).astype(jnp.float32)
    xn_ref[...] = xn * zmask
    sn = sn * zmask
    sn_ref[...] = sn
    tn_ref[...] = _bdot(sn, wgn_ref[...]) * dinv * zmask


def _tc_layer(x, s, t, dinv, px, ps, pt, w1, w2, wgn, bg):
    row_spec = pl.BlockSpec((RBLK, D), lambda i: (i, 0))
    part_spec = pl.BlockSpec((NC, RBLK, D), lambda i: (0, i, 0))
    return pl.pallas_call(
        _tc_layer_body,
        grid=(NP // RBLK,),
        in_specs=[
            row_spec,
            row_spec,
            row_spec,
            pl.BlockSpec((RBLK, 1), lambda i: (i, 0)),
            part_spec,
            part_spec,
            part_spec,
            pl.BlockSpec((2 * D, D), lambda i: (0, 0)),
            pl.BlockSpec((D, D), lambda i: (0, 0)),
            pl.BlockSpec((D, D), lambda i: (0, 0)),
            pl.BlockSpec((1, D), lambda i: (0, 0)),
        ],
        out_specs=[row_spec, row_spec, row_spec],
        out_shape=[
            jax.ShapeDtypeStruct((NP, D), jnp.float32),
            jax.ShapeDtypeStruct((NP, D), jnp.float32),
            jax.ShapeDtypeStruct((NP, D), jnp.float32),
        ],
    )(x, s, t, dinv, px, ps, pt, w1, w2, wgn, bg)


def _tc_final_body(x_ref, s_ref, batch_ref, gamma_ref, beta_ref,
                   whp_ref, bhp_ref, wf1_ref, wf2_ref, wf3_ref, wsc_ref,
                   out_ref):
    x = x_ref[:N, :]
    s = s_ref[:N, :]
    mu = jnp.mean(x, axis=0, keepdims=True)
    xc = x - mu
    var = jnp.mean(xc * xc, axis=0, keepdims=True)
    xl = xc * lax.rsqrt(var + 1e-4) * gamma_ref[...] + beta_ref[...]
    h = _bdot(xl, whp_ref[:D, :]) + _bdot(s, whp_ref[D:, :])
    h = h + bhp_ref[...]
    gids = lax.broadcasted_iota(jnp.int32, (G, 1), 0)
    m = (batch_ref[...] == gids).astype(jnp.float32)
    pooled = jnp.dot(m, h, preferred_element_type=jnp.float32, precision=lax.Precision.HIGHEST)
    b = _leaky(_bdot(pooled, wf1_ref[...]))
    b = _leaky(_bdot(b, wf2_ref[...]))
    b = _leaky(_bdot(b, wf3_ref[...]))
    out_ref[...] = b + _bdot(pooled, wsc_ref[...])


def _tc_final(x, s, batch2d, gamma, beta, whp, bhp, wf1, wf2, wf3, wsc):
    return pl.pallas_call(
        _tc_final_body,
        out_shape=jax.ShapeDtypeStruct((G, D), jnp.float32),
    )(x, s, batch2d, gamma, beta, whp, bhp, wf1, wf2, wf3, wsc)


def kernel(x, edge_index, batch, s, W_pre, b_pre, W_se, b_se, W1, W2,
           gamma, beta, Wg, bg, Whp, b_hp, Wf1, Wf2, Wf3, Wsc):
    src = edge_index[0].reshape(NW, EW)
    dst = edge_index[1].reshape(NW, EW)
    pad = EWP - EW
    srcw = jnp.pad(src, ((0, 0), (0, pad)), constant_values=N)
    dstw = jnp.pad(dst, ((0, 0), (0, pad)), constant_values=N)
    srcw = srcw.reshape(NW, CH, CHW)
    dstw = dstw.reshape(NW, CH, CHW)

    zeros_tab = jnp.zeros((NP, D), jnp.float32)
    ones_tab = jnp.concatenate(
        [jnp.ones((N, D), jnp.float32), jnp.zeros((NP - N, D), jnp.float32)],
        axis=0)

    scat = _get_sc_scatter()
    degp = scat(ones_tab, srcw, dstw, zeros_tab)

    xp = jnp.pad(x, ((0, NP - N), (0, 0)))
    sp = jnp.pad(s, ((0, NP - N), (0, 0)))
    x1, s1, t1, dinv = _tc_pre(
        xp, W_pre, b_pre.reshape(1, D), sp, W_se, b_se.reshape(1, D),
        Wg[0], degp)

    for i in range(LAYERS):
        px = scat(x1, srcw, dstw, zeros_tab)
        ps = scat(s1, srcw, dstw, zeros_tab)
        pt = scat(t1, srcw, dstw, zeros_tab)
        wg_next = Wg[i + 1] if i + 1 < LAYERS else Wg[0]
        x1, s1, t1 = _tc_layer(
            x1, s1, t1, dinv, px, ps, pt, W1[i], W2[i], wg_next,
            bg[i].reshape(1, D))

    return _tc_final(
        x1, s1, batch.reshape(1, N), gamma[2].reshape(1, D),
        beta[2].reshape(1, D), Whp, b_hp.reshape(1, D),
        Wf1, Wf2, Wf3, Wsc)
